# baseline (speedup 1.0000x reference)
import math

import numpy as np
import jax
import jax.numpy as jnp
from jax import lax
from jax.experimental import pallas as pl
from jax.experimental.pallas import tpu as pltpu
from jax.experimental.pallas import tpu_sc as plsc

_H = 16
_NB = 32
_S = 2048
_M = 2 * _S - 1
_GP = 4096
_TW = 3968
_NS = 16
_HPC = 8
_NGRP = 16
_NSPB = 8
_VW = 2944


def _bucket_ids() -> np.ndarray:
    q = np.arange(_M)
    d = (_S - 1) - q
    dist = np.maximum(d, 0)
    small = dist < (_NB // 2)
    ratio = np.log(np.maximum(dist.astype(np.float64), 1.0) / 16.0) / math.log(8.0)
    large = np.minimum(16 + (ratio * 16.0).astype(np.int64), _NB - 1)
    ids = np.where(small, dist, large).astype(np.int32)
    return np.concatenate([ids, np.zeros(_GP - _M, np.int32)])


_BUCKET_IDS = _bucket_ids()


def _sc_body(tab_hbm, bidx_hbm, out_hbm, bidx_v, tab_v, gr_v, tmp_v, v_sh,
             sem, sem_sp):
    cid = lax.axis_index("c")
    tid = lax.axis_index("s")

    pltpu.sync_copy(bidx_hbm, bidx_v)
    pltpu.sync_copy(tab_hbm, tab_v)

    row0 = pl.multiple_of(8 * tid, 8)

    waves = []
    spwaves = []
    for w in range(_HPC):
        h = cid * _HPC + w
        tb = w % 2
        vb = w % 2

        if w >= 2:
            for cp in waves[w - 2]:
                cp.wait()

        hoff = h * _NB

        @plsc.parallel_loop(0, _GP // 16, unroll=8)
        def _gather_body(i):
            b16 = bidx_v[pl.ds(i * 16, 16)]
            gr_v[pl.ds(i * 16, 16)] = plsc.load_gather(tab_v, [hoff + b16])

        for r in range(8):
            off = (127 - r) - 8 * tid

            @plsc.parallel_loop(0, _TW // 16, unroll=8)
            def _shift_body(ci, _r=r, _off=off, _tb=tb):
                tmp_v[_tb, _r, pl.ds(ci * 16, 16)] = gr_v[pl.ds(ci * 16 + _off, 16)]

        cps = []
        for m in range(_NGRP - _NSPB):
            row = pl.multiple_of(8 * tid + 128 * m, 8)
            cp = pltpu.make_async_copy(
                tmp_v.at[tb, :, pl.ds(1920 - 128 * m, _S)],
                out_hbm.at[h, pl.ds(row, 8)],
                sem,
            )
            cp.start()
            cps.append(cp)
        waves.append(cps)

        if w >= 2:
            for i in range(_NSPB):
                @pl.when(tid == i)
                def _wait_sp(_i=i, _w=w):
                    spwaves[_w - 2][_i].wait()
        plsc.subcore_barrier()
        pltpu.sync_copy(tmp_v.at[tb, :, pl.ds(0, _VW)], v_sh.at[vb, pl.ds(row0, 8), :])
        plsc.subcore_barrier()
        spcps = []
        for i in range(_NSPB):
            k = _NGRP - _NSPB + i
            s_k = pl.multiple_of(1920 - 128 * k, 128)
            cp = pltpu.make_async_copy(
                v_sh.at[vb, :, pl.ds(s_k, _S)],
                out_hbm.at[h, pl.ds(pl.multiple_of(128 * k, 8), 128)],
                sem_sp,
            )

            @pl.when(tid == i)
            def _start_sp(_cp=cp):
                _cp.start()

            spcps.append(cp)
        spwaves.append(spcps)

    for w in range(_HPC - 2, _HPC):
        for cp in waves[w]:
            cp.wait()
    for w in range(_HPC - 2, _HPC):
        for i in range(_NSPB):
            @pl.when(tid == i)
            def _drain_sp(_i=i, _w=w):
                spwaves[_w][_i].wait()


def kernel(bias_table, sequence_length):
    del sequence_length
    tab_flat = bias_table.T.reshape(-1)
    bidx = jnp.asarray(_BUCKET_IDS)
    mesh = plsc.VectorSubcoreMesh(
        core_axis_name="c", subcore_axis_name="s", num_cores=2, num_subcores=_NS
    )
    run = pl.kernel(
        _sc_body,
        out_type=jax.ShapeDtypeStruct((_H, _S, _S), jnp.float32),
        mesh=mesh,
        scratch_types=[
            pltpu.VMEM((_GP,), jnp.int32),
            pltpu.VMEM((_H * _NB,), jnp.float32),
            pltpu.VMEM((_GP,), jnp.float32),
            pltpu.VMEM((2, 8, _TW), jnp.float32),
            pltpu.VMEM_SHARED((2, 128, _VW), jnp.float32),
            pltpu.SemaphoreType.DMA,
            pltpu.SemaphoreType.DMA,
        ],
        compiler_params=pltpu.CompilerParams(needs_layout_passes=False),
    )
    return run(tab_flat, bidx)


# speedup vs baseline: 1.0535x; 1.0535x over previous
import math

import numpy as np
import jax
import jax.numpy as jnp
from jax import lax
from jax.experimental import pallas as pl
from jax.experimental.pallas import tpu as pltpu
from jax.experimental.pallas import tpu_sc as plsc

_H = 16
_NB = 32
_S = 2048
_M = 2 * _S - 1
_GP = 4096
_TW = 3968
_NS = 16
_HPC = 8
_NGRP = 16
_NSPB = 7
_VW = 2816


def _bucket_ids() -> np.ndarray:
    q = np.arange(_M)
    d = (_S - 1) - q
    dist = np.maximum(d, 0)
    small = dist < (_NB // 2)
    ratio = np.log(np.maximum(dist.astype(np.float64), 1.0) / 16.0) / math.log(8.0)
    large = np.minimum(16 + (ratio * 16.0).astype(np.int64), _NB - 1)
    ids = np.where(small, dist, large).astype(np.int32)
    return np.concatenate([ids, np.zeros(_GP - _M, np.int32)])


_BUCKET_IDS = _bucket_ids()


def _sc_body(tab_hbm, bidx_hbm, out_hbm, bidx_v, tab_v, gr_v, tmp_v, v_sh,
             sem, sem_sp):
    cid = lax.axis_index("c")
    tid = lax.axis_index("s")

    pltpu.sync_copy(bidx_hbm, bidx_v)
    pltpu.sync_copy(tab_hbm, tab_v)

    row0 = pl.multiple_of(8 * tid, 8)

    waves = []
    spwaves = []
    for w in range(_HPC):
        h = cid * _HPC + w
        tb = w % 2
        vb = w % 2

        if w >= 2:
            for cp in waves[w - 2]:
                cp.wait()

        hoff = h * _NB

        @plsc.parallel_loop(0, _GP // 16, unroll=8)
        def _gather_body(i):
            b16 = bidx_v[pl.ds(i * 16, 16)]
            gr_v[pl.ds(i * 16, 16)] = plsc.load_gather(tab_v, [hoff + b16])

        for r in range(8):
            off = (127 - r) - 8 * tid

            @plsc.parallel_loop(0, _TW // 16, unroll=8)
            def _shift_body(ci, _r=r, _off=off, _tb=tb):
                tmp_v[_tb, _r, pl.ds(ci * 16, 16)] = gr_v[pl.ds(ci * 16 + _off, 16)]

        cps = []
        for m in range(_NGRP - _NSPB):
            row = pl.multiple_of(8 * tid + 128 * m, 8)
            cp = pltpu.make_async_copy(
                tmp_v.at[tb, :, pl.ds(1920 - 128 * m, _S)],
                out_hbm.at[h, pl.ds(row, 8)],
                sem,
            )
            cp.start()
            cps.append(cp)
        waves.append(cps)

        if w >= 2:
            for i in range(_NSPB):
                @pl.when(tid == i)
                def _wait_sp(_i=i, _w=w):
                    spwaves[_w - 2][_i].wait()
        plsc.subcore_barrier()
        pltpu.sync_copy(tmp_v.at[tb, :, pl.ds(0, _VW)], v_sh.at[vb, pl.ds(row0, 8), :])
        plsc.subcore_barrier()
        spcps = []
        for i in range(_NSPB):
            k = _NGRP - _NSPB + i
            s_k = pl.multiple_of(1920 - 128 * k, 128)
            cp = pltpu.make_async_copy(
                v_sh.at[vb, :, pl.ds(s_k, _S)],
                out_hbm.at[h, pl.ds(pl.multiple_of(128 * k, 8), 128)],
                sem_sp,
            )

            @pl.when(tid == i)
            def _start_sp(_cp=cp):
                _cp.start()

            spcps.append(cp)
        spwaves.append(spcps)

    for w in range(_HPC - 2, _HPC):
        for cp in waves[w]:
            cp.wait()
    for w in range(_HPC - 2, _HPC):
        for i in range(_NSPB):
            @pl.when(tid == i)
            def _drain_sp(_i=i, _w=w):
                spwaves[_w][_i].wait()


def kernel(bias_table, sequence_length):
    del sequence_length
    tab_flat = bias_table.T.reshape(-1)
    bidx = jnp.asarray(_BUCKET_IDS)
    mesh = plsc.VectorSubcoreMesh(
        core_axis_name="c", subcore_axis_name="s", num_cores=2, num_subcores=_NS
    )
    run = pl.kernel(
        _sc_body,
        out_type=jax.ShapeDtypeStruct((_H, _S, _S), jnp.float32),
        mesh=mesh,
        scratch_types=[
            pltpu.VMEM((_GP,), jnp.int32),
            pltpu.VMEM((_H * _NB,), jnp.float32),
            pltpu.VMEM((_GP,), jnp.float32),
            pltpu.VMEM((2, 8, _TW), jnp.float32),
            pltpu.VMEM_SHARED((2, 128, _VW), jnp.float32),
            pltpu.SemaphoreType.DMA,
            pltpu.SemaphoreType.DMA,
        ],
        compiler_params=pltpu.CompilerParams(needs_layout_passes=False),
    )
    return run(tab_flat, bidx)
